# baseline (speedup 1.0000x reference)
import jax
import jax.numpy as jnp
from jax import lax
from jax.experimental import pallas as pl
from jax.experimental.pallas import tpu as pltpu
from jax.experimental.pallas import tpu_sc as plsc

_VOCAB = 100000
_EMBED = 64
_CTX = 20
_BATCH = 1024
_HIDDEN = 128

_NC = 2
_NS = 16
_NW = _NC * _NS
_BTOT = _BATCH * _CTX
_BPW = _BTOT // _NW
_CH = 128
_NCH = _BPW // _CH

_VB = 2048
_NJ = (_VOCAB + _VB - 1) // _VB


def _gather_body(table_hbm, idx_hbm, out_hbm, idx_v, rows_v, sem):
    wid = lax.axis_index("s") * _NC + lax.axis_index("c")
    base = wid * _BPW
    pltpu.sync_copy(idx_hbm.at[wid], idx_v)
    copies = []
    for i in range(_NCH):
        c = pltpu.make_async_copy(
            table_hbm.at[idx_v.at[i]],
            rows_v.at[pl.ds(i * _CH, _CH)],
            sem,
        )
        c.start()
        copies.append(c)
    for c in copies:
        c.wait()
    pltpu.sync_copy(rows_v, out_hbm.at[pl.ds(base, _BPW)])


def _sc_gather(emb_table, idx):
    mesh = plsc.VectorSubcoreMesh(
        core_axis_name="c", subcore_axis_name="s",
        num_cores=_NC, num_subcores=_NS,
    )
    return pl.kernel(
        _gather_body,
        out_type=jax.ShapeDtypeStruct((_BTOT, _EMBED), jnp.float32),
        mesh=mesh,
        scratch_types=[
            pltpu.VMEM((_NCH, _CH), jnp.int32),
            pltpu.VMEM((_BPW, _EMBED), jnp.float32),
            pltpu.SemaphoreType.DMA,
        ],
        compiler_params=pltpu.CompilerParams(use_tc_tiling_on_sc=False),
    )(emb_table, idx)


def _mlp_body(embeds, W1, b1, W2, b2, out, h_ref, m_ref, s_ref):
    p = pl.program_id(0)
    j = pl.program_id(1)

    @pl.when((p == 0) & (j == 0))
    def _():
        pre = jnp.dot(embeds[...], W1[...], preferred_element_type=jnp.float32)
        pre = pre + b1[...]
        h_ref[...] = jnp.maximum(pre, 0.0).astype(jnp.bfloat16)
        m_ref[...] = jnp.full(m_ref.shape, -jnp.inf, jnp.float32)
        s_ref[...] = jnp.zeros(s_ref.shape, jnp.float32)

    logits = jnp.dot(h_ref[...], W2[...], preferred_element_type=jnp.float32)
    logits = logits + b2[...]
    col_ok = (j * _VB + lax.broadcasted_iota(jnp.int32, (1, _VB), 1)) < _VOCAB

    @pl.when(p == 0)
    def _():
        lm = jnp.where(col_ok, logits, -jnp.inf)
        bm = jnp.max(lm, axis=1, keepdims=True)
        new_m = jnp.maximum(m_ref[...], bm)
        s_ref[...] = s_ref[...] * jnp.exp(m_ref[...] - new_m) + jnp.sum(
            jnp.exp(lm - new_m), axis=1, keepdims=True)
        m_ref[...] = new_m

    @pl.when(p == 1)
    def _():
        out[...] = logits - (m_ref[...] + jnp.log(s_ref[...]))


def _mlp_logsoftmax(embeds, W1, b1, W2, b2):
    return pl.pallas_call(
        _mlp_body,
        grid=(2, _NJ),
        in_specs=[
            pl.BlockSpec((_BATCH, _CTX * _EMBED), lambda p, j: (0, 0)),
            pl.BlockSpec((_CTX * _EMBED, _HIDDEN), lambda p, j: (0, 0)),
            pl.BlockSpec((1, _HIDDEN), lambda p, j: (0, 0)),
            pl.BlockSpec((_HIDDEN, _VB), lambda p, j: (0, j)),
            pl.BlockSpec((1, _VB), lambda p, j: (0, j)),
        ],
        out_specs=pl.BlockSpec((_BATCH, _VB), lambda p, j: (0, j * p)),
        out_shape=jax.ShapeDtypeStruct((_BATCH, _VOCAB), jnp.float32),
        scratch_shapes=[
            pltpu.VMEM((_BATCH, _HIDDEN), jnp.bfloat16),
            pltpu.VMEM((_BATCH, 1), jnp.float32),
            pltpu.VMEM((_BATCH, 1), jnp.float32),
        ],
    )(embeds, W1, b1, W2, b2)


def kernel(inputs, emb_table, W1, b1, W2, b2):
    idx = inputs.reshape(_NW, _NCH, _CH)
    embeds = _sc_gather(emb_table, idx)
    embeds = embeds.reshape(_BATCH, _CTX * _EMBED).astype(jnp.bfloat16)
    return _mlp_logsoftmax(
        embeds,
        W1.astype(jnp.bfloat16),
        b1.reshape(1, _HIDDEN),
        W2.astype(jnp.bfloat16),
        b2.reshape(1, _VOCAB),
    )


# speedup vs baseline: 1.0295x; 1.0295x over previous
import jax
import jax.numpy as jnp
from jax import lax
from jax.experimental import pallas as pl
from jax.experimental.pallas import tpu as pltpu
from jax.experimental.pallas import tpu_sc as plsc

_VOCAB = 100000
_EMBED = 64
_CTX = 20
_BATCH = 1024
_HIDDEN = 128
_KAUG = 136

_NC = 2
_NS = 16
_NW = _NC * _NS
_BTOT = _BATCH * _CTX
_BPW = _BTOT // _NW
_CH = 128
_NCH = _BPW // _CH

_VB = 2048
_NJ = (_VOCAB + _VB - 1) // _VB
_VPAD = _NJ * _VB


def _gather_body(table_hbm, idx_hbm, out_hbm, idx_v, rows_v, sem):
    wid = lax.axis_index("s") * _NC + lax.axis_index("c")
    base = wid * _BPW
    pltpu.sync_copy(idx_hbm.at[wid], idx_v)
    copies = []
    for i in range(_NCH):
        c = pltpu.make_async_copy(
            table_hbm.at[idx_v.at[i]],
            rows_v.at[pl.ds(i * _CH, _CH)],
            sem,
        )
        c.start()
        copies.append(c)
    for c in copies:
        c.wait()
    pltpu.sync_copy(rows_v, out_hbm.at[pl.ds(base, _BPW)])


def _sc_gather(emb_table, idx):
    mesh = plsc.VectorSubcoreMesh(
        core_axis_name="c", subcore_axis_name="s",
        num_cores=_NC, num_subcores=_NS,
    )
    return pl.kernel(
        _gather_body,
        out_type=jax.ShapeDtypeStruct((_BTOT, _EMBED), jnp.float32),
        mesh=mesh,
        scratch_types=[
            pltpu.VMEM((_NCH, _CH), jnp.int32),
            pltpu.VMEM((_BPW, _EMBED), jnp.float32),
            pltpu.SemaphoreType.DMA,
        ],
        compiler_params=pltpu.CompilerParams(use_tc_tiling_on_sc=False),
    )(emb_table, idx)


def _mlp_body(embeds, W1, b1, W2a, out, h_ref, m_ref, s_ref):
    p = pl.program_id(0)
    j = pl.program_id(1)

    @pl.when((p == 0) & (j == 0))
    def _():
        pre = jnp.dot(embeds[...], W1[...], preferred_element_type=jnp.float32)
        pre = pre + b1[...]
        h_ref[:, 0:_HIDDEN] = jnp.maximum(pre, 0.0).astype(jnp.bfloat16)
        lane = lax.broadcasted_iota(jnp.int32, (_BATCH, 8), 1)
        ext = jnp.where(lane == 0, 1.0, 0.0)
        h_ref[:, _HIDDEN:_KAUG] = ext.astype(jnp.bfloat16)
        m_ref[...] = jnp.full(m_ref.shape, -jnp.inf, jnp.float32)
        s_ref[...] = jnp.zeros(s_ref.shape, jnp.float32)

    @pl.when((p == 1) & (j == 0))
    def _():
        lse = m_ref[...] + jnp.log(s_ref[...])
        hi = lse.astype(jnp.bfloat16)
        lo = (lse - hi.astype(jnp.float32)).astype(jnp.bfloat16)
        h_ref[:, _HIDDEN + 1:_HIDDEN + 3] = jnp.concatenate([hi, lo], axis=1)

    logits = jnp.dot(h_ref[...], W2a[...], preferred_element_type=jnp.float32)

    @pl.when(p == 0)
    def _():
        bm = jnp.max(logits, axis=1, keepdims=True)
        new_m = jnp.maximum(m_ref[...], bm)
        s_ref[...] = s_ref[...] * jnp.exp(m_ref[...] - new_m) + jnp.sum(
            jnp.exp(logits - new_m), axis=1, keepdims=True)
        m_ref[...] = new_m

    @pl.when(p == 1)
    def _():
        out[...] = logits


def _mlp_logsoftmax(embeds, W1, b1, W2a):
    return pl.pallas_call(
        _mlp_body,
        grid=(2, _NJ),
        in_specs=[
            pl.BlockSpec((_BATCH, _CTX * _EMBED), lambda p, j: (0, 0)),
            pl.BlockSpec((_CTX * _EMBED, _HIDDEN), lambda p, j: (0, 0)),
            pl.BlockSpec((1, _HIDDEN), lambda p, j: (0, 0)),
            pl.BlockSpec((_KAUG, _VB), lambda p, j: (0, j)),
        ],
        out_specs=pl.BlockSpec((_BATCH, _VB), lambda p, j: (0, j * p)),
        out_shape=jax.ShapeDtypeStruct((_BATCH, _VOCAB), jnp.float32),
        scratch_shapes=[
            pltpu.VMEM((_BATCH, _KAUG), jnp.bfloat16),
            pltpu.VMEM((_BATCH, 1), jnp.float32),
            pltpu.VMEM((_BATCH, 1), jnp.float32),
        ],
    )(embeds, W1, b1, W2a)


def _augment_w2(W2, b2):
    pad = _VPAD - _VOCAB
    w2b = jnp.pad(W2.astype(jnp.bfloat16), ((0, 0), (0, pad)))
    b2b = jnp.pad(b2.reshape(1, _VOCAB).astype(jnp.bfloat16),
                  ((0, 0), (0, pad)), constant_values=jnp.bfloat16(-1e30))
    ones = jnp.full((2, _VPAD), -1.0, jnp.bfloat16)
    zeros = jnp.zeros((_KAUG - _HIDDEN - 3, _VPAD), jnp.bfloat16)
    return jnp.concatenate([w2b, b2b, ones, zeros], axis=0)


def kernel(inputs, emb_table, W1, b1, W2, b2):
    idx = inputs.reshape(_NW, _NCH, _CH)
    embeds = _sc_gather(emb_table, idx)
    embeds = embeds.reshape(_BATCH, _CTX * _EMBED).astype(jnp.bfloat16)
    return _mlp_logsoftmax(
        embeds,
        W1.astype(jnp.bfloat16),
        b1.reshape(1, _HIDDEN),
        _augment_w2(W2, b2),
    )


# speedup vs baseline: 1.0365x; 1.0068x over previous
import jax
import jax.numpy as jnp
from jax import lax
from jax.experimental import pallas as pl
from jax.experimental.pallas import tpu as pltpu
from jax.experimental.pallas import tpu_sc as plsc

_VOCAB = 100000
_EMBED = 64
_EPAD = 128
_CTX = 20
_BATCH = 1024
_HIDDEN = 128
_KAUG = 136

_NC = 2
_NS = 16
_NW = _NC * _NS
_BTOT = _BATCH * _CTX
_BPW = _BTOT // _NW
_CH = 128
_NCH = _BPW // _CH

_VB = 2048
_NJ = (_VOCAB + _VB - 1) // _VB
_VPAD = _NJ * _VB


def _gather_body(table_hbm, idx_hbm, out_hbm, idx_v, rows_v, sem):
    wid = lax.axis_index("s") * _NC + lax.axis_index("c")
    base = wid * _BPW
    pltpu.sync_copy(idx_hbm.at[pl.ds(base, _BPW)], idx_v)
    copies = []
    for i in range(_NCH):
        c = pltpu.make_async_copy(
            table_hbm.at[idx_v.at[pl.ds(i * _CH, _CH)]],
            rows_v.at[pl.ds(i * _CH, _CH)],
            sem,
        )
        c.start()
        copies.append(c)
    for c in copies:
        c.wait()
    pltpu.sync_copy(rows_v, out_hbm.at[pl.ds(base, _BPW)])


def _sc_gather(table128, idx):
    mesh = plsc.VectorSubcoreMesh(
        core_axis_name="c", subcore_axis_name="s",
        num_cores=_NC, num_subcores=_NS,
    )
    return pl.kernel(
        _gather_body,
        out_type=jax.ShapeDtypeStruct((_BTOT, _EPAD), jnp.float32),
        mesh=mesh,
        scratch_types=[
            pltpu.VMEM((_BPW,), jnp.int32),
            pltpu.VMEM((_BPW, _EPAD), jnp.float32),
            pltpu.SemaphoreType.DMA,
        ],
    )(table128, idx)


def _mlp_body(embeds, W1, b1, W2a, out, h_ref, m_ref, s_ref):
    p = pl.program_id(0)
    j = pl.program_id(1)

    @pl.when((p == 0) & (j == 0))
    def _():
        pre = jnp.dot(embeds[...], W1[...], preferred_element_type=jnp.float32)
        pre = pre + b1[...]
        h_ref[:, 0:_HIDDEN] = jnp.maximum(pre, 0.0).astype(jnp.bfloat16)
        lane = lax.broadcasted_iota(jnp.int32, (_BATCH, 8), 1)
        ext = jnp.where(lane == 0, 1.0, 0.0)
        h_ref[:, _HIDDEN:_KAUG] = ext.astype(jnp.bfloat16)
        m_ref[...] = jnp.full(m_ref.shape, -jnp.inf, jnp.float32)
        s_ref[...] = jnp.zeros(s_ref.shape, jnp.float32)

    @pl.when((p == 1) & (j == 0))
    def _():
        lse = m_ref[...] + jnp.log(s_ref[...])
        hi = lse.astype(jnp.bfloat16)
        lo = (lse - hi.astype(jnp.float32)).astype(jnp.bfloat16)
        h_ref[:, _HIDDEN + 1:_HIDDEN + 3] = jnp.concatenate([hi, lo], axis=1)

    logits = jnp.dot(h_ref[...], W2a[...], preferred_element_type=jnp.float32)

    @pl.when(p == 0)
    def _():
        bm = jnp.max(logits, axis=1, keepdims=True)
        new_m = jnp.maximum(m_ref[...], bm)
        s_ref[...] = s_ref[...] * jnp.exp(m_ref[...] - new_m) + jnp.sum(
            jnp.exp(logits - new_m), axis=1, keepdims=True)
        m_ref[...] = new_m

    @pl.when(p == 1)
    def _():
        out[...] = logits


def _mlp_logsoftmax(embeds, W1, b1, W2a):
    return pl.pallas_call(
        _mlp_body,
        grid=(2, _NJ),
        in_specs=[
            pl.BlockSpec((_BATCH, _CTX * _EPAD), lambda p, j: (0, 0)),
            pl.BlockSpec((_CTX * _EPAD, _HIDDEN), lambda p, j: (0, 0)),
            pl.BlockSpec((1, _HIDDEN), lambda p, j: (0, 0)),
            pl.BlockSpec((_KAUG, _VB), lambda p, j: (0, j)),
        ],
        out_specs=pl.BlockSpec((_BATCH, _VB), lambda p, j: (0, j * p)),
        out_shape=jax.ShapeDtypeStruct((_BATCH, _VOCAB), jnp.float32),
        scratch_shapes=[
            pltpu.VMEM((_BATCH, _KAUG), jnp.bfloat16),
            pltpu.VMEM((_BATCH, 1), jnp.float32),
            pltpu.VMEM((_BATCH, 1), jnp.float32),
        ],
    )(embeds, W1, b1, W2a)


def _augment_w2(W2, b2):
    pad = _VPAD - _VOCAB
    w2b = jnp.pad(W2.astype(jnp.bfloat16), ((0, 0), (0, pad)))
    b2b = jnp.pad(b2.reshape(1, _VOCAB).astype(jnp.bfloat16),
                  ((0, 0), (0, pad)), constant_values=jnp.bfloat16(-1e30))
    ones = jnp.full((2, _VPAD), -1.0, jnp.bfloat16)
    zeros = jnp.zeros((_KAUG - _HIDDEN - 3, _VPAD), jnp.bfloat16)
    return jnp.concatenate([w2b, b2b, ones, zeros], axis=0)


def _widen_w1(W1):
    w1 = W1.astype(jnp.bfloat16).reshape(_CTX, _EMBED, _HIDDEN)
    w1 = jnp.pad(w1, ((0, 0), (0, _EPAD - _EMBED), (0, 0)))
    return w1.reshape(_CTX * _EPAD, _HIDDEN)


def kernel(inputs, emb_table, W1, b1, W2, b2):
    idx = inputs.reshape(_BTOT)
    table128 = jnp.pad(emb_table, ((0, 0), (0, _EPAD - _EMBED)))
    embeds = _sc_gather(table128, idx)
    embeds = embeds.astype(jnp.bfloat16).reshape(_BATCH, _CTX * _EPAD)
    return _mlp_logsoftmax(
        embeds,
        _widen_w1(W1),
        b1.reshape(1, _HIDDEN),
        _augment_w2(W2, b2),
    )


# speedup vs baseline: 1.3025x; 1.2567x over previous
import jax
import jax.numpy as jnp
from jax import lax
from jax.experimental import pallas as pl
from jax.experimental.pallas import tpu as pltpu
from jax.experimental.pallas import tpu_sc as plsc

_VOCAB = 100000
_EMBED = 64
_EPAD = 128
_CTX = 20
_BATCH = 1024
_HIDDEN = 128
_KAUG = 136

_NC = 2
_NS = 16
_NW = _NC * _NS
_BTOT = _BATCH * _CTX
_BPW = _BTOT // _NW
_CH = 128
_NCH = _BPW // _CH

_VB = 2048
_NJ = (_VOCAB + _VB - 1) // _VB
_VPAD = _NJ * _VB


def _gather_body(table_hbm, idx_hbm, out_hbm, idx_v, rows_v, sem):
    wid = lax.axis_index("s") * _NC + lax.axis_index("c")
    base = wid * _BPW
    pltpu.sync_copy(idx_hbm.at[pl.ds(base, _BPW)], idx_v)
    copies = []
    for i in range(_NCH):
        c = pltpu.make_async_copy(
            table_hbm.at[idx_v.at[pl.ds(i * _CH, _CH)]],
            rows_v.at[pl.ds(i * _CH, _CH)],
            sem,
        )
        c.start()
        copies.append(c)
    for c in copies:
        c.wait()
    pltpu.sync_copy(rows_v, out_hbm.at[pl.ds(base, _BPW)])


def _sc_gather(table128, idx):
    mesh = plsc.VectorSubcoreMesh(
        core_axis_name="c", subcore_axis_name="s",
        num_cores=_NC, num_subcores=_NS,
    )
    return pl.kernel(
        _gather_body,
        out_type=jax.ShapeDtypeStruct((_BTOT, _EPAD), jnp.float32),
        mesh=mesh,
        scratch_types=[
            pltpu.VMEM((_BPW,), jnp.int32),
            pltpu.VMEM((_BPW, _EPAD), jnp.float32),
            pltpu.SemaphoreType.DMA,
        ],
    )(table128, idx)


def _mlp_body(embT, W1T, b1c, W2aT, out, h_ref, m_ref, s_ref):
    p = pl.program_id(0)
    j = pl.program_id(1)

    @pl.when((p == 0) & (j == 0))
    def _():
        pre = jnp.dot(W1T[...], embT[...], preferred_element_type=jnp.float32)
        pre = pre + b1c[...]
        h_ref[0:_HIDDEN, :] = jnp.maximum(pre, 0.0).astype(jnp.bfloat16)
        row = lax.broadcasted_iota(jnp.int32, (8, _BATCH), 0)
        ext = jnp.where(row == 0, 1.0, 0.0)
        h_ref[_HIDDEN:_KAUG, :] = ext.astype(jnp.bfloat16)
        m_ref[...] = jnp.full(m_ref.shape, -jnp.inf, jnp.float32)
        s_ref[...] = jnp.zeros(s_ref.shape, jnp.float32)

    @pl.when((p == 1) & (j == 0))
    def _():
        lse = m_ref[...] + jnp.log(s_ref[...])
        hi = lse.astype(jnp.bfloat16)
        lo = (lse - hi.astype(jnp.float32)).astype(jnp.bfloat16)
        h_ref[_HIDDEN + 1:_HIDDEN + 3, :] = jnp.concatenate([hi, lo], axis=0)

    logits = jnp.dot(W2aT[...], h_ref[...], preferred_element_type=jnp.float32)

    @pl.when(p == 0)
    def _():
        bm = jnp.max(logits, axis=0, keepdims=True)
        new_m = jnp.maximum(m_ref[...], bm)
        s_ref[...] = s_ref[...] * jnp.exp(m_ref[...] - new_m) + jnp.sum(
            jnp.exp(logits - new_m), axis=0, keepdims=True)
        m_ref[...] = new_m

    @pl.when(p == 1)
    def _():
        out[...] = logits


def _mlp_logsoftmax_t(embT, W1T, b1c, W2aT):
    return pl.pallas_call(
        _mlp_body,
        grid=(2, _NJ),
        in_specs=[
            pl.BlockSpec((_CTX * _EPAD, _BATCH), lambda p, j: (0, 0)),
            pl.BlockSpec((_HIDDEN, _CTX * _EPAD), lambda p, j: (0, 0)),
            pl.BlockSpec((_HIDDEN, 1), lambda p, j: (0, 0)),
            pl.BlockSpec((_VB, _KAUG), lambda p, j: (j, 0)),
        ],
        out_specs=pl.BlockSpec((_VB, _BATCH), lambda p, j: (j * p, 0)),
        out_shape=jax.ShapeDtypeStruct((_VOCAB, _BATCH), jnp.float32),
        scratch_shapes=[
            pltpu.VMEM((_KAUG, _BATCH), jnp.bfloat16),
            pltpu.VMEM((1, _BATCH), jnp.float32),
            pltpu.VMEM((1, _BATCH), jnp.float32),
        ],
    )(embT, W1T, b1c, W2aT)


def _augment_w2_t(W2, b2):
    w2t = W2.astype(jnp.bfloat16).T
    b2c = b2.reshape(_VOCAB, 1).astype(jnp.bfloat16)
    ones2 = jnp.full((_VOCAB, 2), -1.0, jnp.bfloat16)
    zer5 = jnp.zeros((_VOCAB, _KAUG - _HIDDEN - 3), jnp.bfloat16)
    top = jnp.concatenate([w2t, b2c, ones2, zer5], axis=1)
    npad = _VPAD - _VOCAB
    padrows = jnp.concatenate([
        jnp.zeros((npad, _HIDDEN), jnp.bfloat16),
        jnp.full((npad, 1), -1e30, jnp.bfloat16),
        jnp.full((npad, 2), -1.0, jnp.bfloat16),
        jnp.zeros((npad, _KAUG - _HIDDEN - 3), jnp.bfloat16),
    ], axis=1)
    return jnp.concatenate([top, padrows], axis=0)


def _widen_w1_t(W1):
    w1 = W1.astype(jnp.bfloat16).reshape(_CTX, _EMBED, _HIDDEN)
    w1 = jnp.pad(w1, ((0, 0), (0, _EPAD - _EMBED), (0, 0)))
    return w1.reshape(_CTX * _EPAD, _HIDDEN).T


def kernel(inputs, emb_table, W1, b1, W2, b2):
    idx = inputs.reshape(_BTOT)
    table128 = jnp.pad(emb_table, ((0, 0), (0, _EPAD - _EMBED)))
    embeds = _sc_gather(table128, idx)
    embT = embeds.astype(jnp.bfloat16).reshape(_BATCH, _CTX * _EPAD).T
    outT = _mlp_logsoftmax_t(
        embT,
        _widen_w1_t(W1),
        b1.reshape(_HIDDEN, 1),
        _augment_w2_t(W2, b2),
    )
    return outT.T


# speedup vs baseline: 1.7878x; 1.3725x over previous
import jax
import jax.numpy as jnp
from jax import lax
from jax.experimental import pallas as pl
from jax.experimental.pallas import tpu as pltpu
from jax.experimental.pallas import tpu_sc as plsc

_VOCAB = 100000
_EMBED = 64
_EPAD = 128
_CTX = 20
_BATCH = 1024
_HIDDEN = 128
_KAUG = 136

_NC = 2
_NS = 16
_NW = _NC * _NS
_BTOT = _BATCH * _CTX
_BPW = _BTOT // _NW
_CH = 128
_NCH = _BPW // _CH

_VB = 2048
_NJ = (_VOCAB + _VB - 1) // _VB
_VPAD = _NJ * _VB


def _gather_body(table_hbm, idx_hbm, out_hbm, idx_v, rows_v, sem):
    wid = lax.axis_index("s") * _NC + lax.axis_index("c")
    base = wid * _BPW
    pltpu.sync_copy(idx_hbm.at[pl.ds(base, _BPW)], idx_v)
    copies = []
    for i in range(_NCH):
        c = pltpu.make_async_copy(
            table_hbm.at[idx_v.at[pl.ds(i * _CH, _CH)]],
            rows_v.at[pl.ds(i * _CH, _CH)],
            sem,
        )
        c.start()
        copies.append(c)
    for c in copies:
        c.wait()
    pltpu.sync_copy(rows_v, out_hbm.at[pl.ds(base, _BPW)])


def _sc_gather(table128, idx):
    mesh = plsc.VectorSubcoreMesh(
        core_axis_name="c", subcore_axis_name="s",
        num_cores=_NC, num_subcores=_NS,
    )
    return pl.kernel(
        _gather_body,
        out_type=jax.ShapeDtypeStruct((_BTOT, _EPAD), jnp.float32),
        mesh=mesh,
        scratch_types=[
            pltpu.VMEM((_BPW,), jnp.int32),
            pltpu.VMEM((_BPW, _EPAD), jnp.float32),
            pltpu.SemaphoreType.DMA,
        ],
    )(table128, idx)


def _mlp_body(embT, W1T, b1c, W2aT, out, h_ref, m_ref, s_ref):
    p = pl.program_id(0)
    j = pl.program_id(1)

    @pl.when((p == 0) & (j == 0))
    def _():
        pre = jnp.dot(W1T[...], embT[...], preferred_element_type=jnp.float32)
        pre = pre + b1c[...]
        h_ref[0:_HIDDEN, :] = jnp.maximum(pre, 0.0).astype(jnp.bfloat16)
        row = lax.broadcasted_iota(jnp.int32, (8, _BATCH), 0)
        ext = jnp.where(row == 0, 1.0, 0.0)
        h_ref[_HIDDEN:_KAUG, :] = ext.astype(jnp.bfloat16)
        m_ref[...] = jnp.full(m_ref.shape, -jnp.inf, jnp.float32)
        s_ref[...] = jnp.zeros(s_ref.shape, jnp.float32)

    @pl.when((p == 1) & (j == 0))
    def _():
        lse = m_ref[...] + jnp.log(s_ref[...])
        hi = lse.astype(jnp.bfloat16)
        lo = (lse - hi.astype(jnp.float32)).astype(jnp.bfloat16)
        h_ref[_HIDDEN + 1:_HIDDEN + 3, :] = jnp.concatenate([hi, lo], axis=0)

    logits = lax.dot_general(
        W2aT[...], h_ref[...],
        dimension_numbers=(((0,), (0,)), ((), ())),
        preferred_element_type=jnp.float32)

    @pl.when(p == 0)
    def _():
        bm = jnp.max(logits, axis=0, keepdims=True)
        new_m = jnp.maximum(m_ref[...], bm)
        s_ref[...] = s_ref[...] * jnp.exp(m_ref[...] - new_m) + jnp.sum(
            jnp.exp(logits - new_m), axis=0, keepdims=True)
        m_ref[...] = new_m

    @pl.when(p == 1)
    def _():
        out[...] = logits


def _mlp_logsoftmax_t(embT, W1T, b1c, W2aT):
    return pl.pallas_call(
        _mlp_body,
        grid=(2, _NJ),
        in_specs=[
            pl.BlockSpec((_CTX * _EPAD, _BATCH), lambda p, j: (0, 0)),
            pl.BlockSpec((_HIDDEN, _CTX * _EPAD), lambda p, j: (0, 0)),
            pl.BlockSpec((_HIDDEN, 1), lambda p, j: (0, 0)),
            pl.BlockSpec((_KAUG, _VB), lambda p, j: (0, j)),
        ],
        out_specs=pl.BlockSpec((_VB, _BATCH), lambda p, j: (j * p, 0)),
        out_shape=jax.ShapeDtypeStruct((_VOCAB, _BATCH), jnp.float32),
        scratch_shapes=[
            pltpu.VMEM((_KAUG, _BATCH), jnp.bfloat16),
            pltpu.VMEM((1, _BATCH), jnp.float32),
            pltpu.VMEM((1, _BATCH), jnp.float32),
        ],
    )(embT, W1T, b1c, W2aT)


def _augment_w2_t(W2, b2):
    npad = _VPAD - _VOCAB
    w2b = jnp.pad(W2.astype(jnp.bfloat16), ((0, 0), (0, npad)))
    b2row = jnp.pad(b2.reshape(1, _VOCAB).astype(jnp.bfloat16),
                    ((0, 0), (0, npad)), constant_values=-1e30)
    ones2 = jnp.full((2, _VPAD), -1.0, jnp.bfloat16)
    zer5 = jnp.zeros((_KAUG - _HIDDEN - 3, _VPAD), jnp.bfloat16)
    return jnp.concatenate([w2b, b2row, ones2, zer5], axis=0)


def _widen_w1_t(W1):
    w1 = W1.astype(jnp.bfloat16).reshape(_CTX, _EMBED, _HIDDEN)
    w1 = jnp.pad(w1, ((0, 0), (0, _EPAD - _EMBED), (0, 0)))
    return w1.reshape(_CTX * _EPAD, _HIDDEN).T


def kernel(inputs, emb_table, W1, b1, W2, b2):
    idx = inputs.reshape(_BTOT)
    table128 = jnp.pad(emb_table, ((0, 0), (0, _EPAD - _EMBED)))
    embeds = _sc_gather(table128, idx)
    embT = embeds.astype(jnp.bfloat16).reshape(_BATCH, _CTX * _EPAD).T
    outT = _mlp_logsoftmax_t(
        embT,
        _widen_w1_t(W1),
        b1.reshape(_HIDDEN, 1),
        _augment_w2_t(W2, b2),
    )
    return outT.T
